# baseline (speedup 1.0000x reference)
import functools

import jax
import jax.numpy as jnp
from jax import lax
from jax.experimental import pallas as pl
from jax.experimental.pallas import tpu as pltpu
from jax.experimental.pallas import tpu_sc as plsc

VOCAB = 1000000
D = 64
B = 4096
L = 200
N = B * L
LANES = 16
BB = 64
LC = 4
CHUNK = LC * BB
NCHUNKS = L // LC
EPS = 1e-5


def _positional_encoding(d_model, max_len):
    pos = jnp.arange(max_len, dtype=jnp.float32)[:, None]
    div = jnp.exp(
        jnp.arange(0, d_model, 2, dtype=jnp.float32) * (-jnp.log(10000.0) / d_model)
    )
    ang = pos * div[None, :]
    pe = jnp.zeros((max_len, d_model), dtype=jnp.float32)
    pe = pe.at[:, 0::2].set(jnp.sin(ang))
    pe = pe.at[:, 1::2].set(jnp.cos(ang))
    return pe


def _rsqrt_sc(x):
    i = lax.bitcast_convert_type(x, jnp.int32)
    i = jnp.int32(0x5F3759DF) - (i >> 1)
    y = lax.bitcast_convert_type(i, jnp.float32)
    for _ in range(2):
        y = y * (1.5 - 0.5 * x * y * y)
    return y


def _sc_body(nc, ids_hbm, tags_hbm, table_hbm, comb_hbm, gam_hbm, bet_hbm,
             out_hbm, ids_v, tags_v, idx_v, rows_v, comb_v, gam_v, bet_v,
             tbuf_v, out_v, sem_g, sem_w):
    wid = lax.axis_index("s") * nc + lax.axis_index("c")

    pltpu.sync_copy(comb_hbm, comb_v)
    pltpu.sync_copy(gam_hbm, gam_v)
    pltpu.sync_copy(bet_hbm, bet_v)

    gvs = [gam_v[pl.ds(k * LANES, LANES)] for k in range(D // LANES)]
    bvs = [bet_v[pl.ds(k * LANES, LANES)] for k in range(D // LANES)]

    lanes = lax.iota(jnp.int32, LANES)
    j200 = lanes * L

    def bcast(v, j):
        return v.at[jnp.full((LANES,), j, jnp.int32)].get(mode="promise_in_bounds")

    def issue_gather(par):
        pltpu.async_copy(table_hbm.at[idx_v[par]], rows_v[par], sem_g[par])

    def wait_gather(par):
        pltpu.make_async_copy(table_hbm.at[idx_v[par]], rows_v[par], sem_g[par]).wait()

    def build_idx(ci, par):
        l0 = ci * LC
        for li in range(LC):
            for jg in range(BB // LANES):
                src = j200 + (jg * LANES * L + l0 + li)
                val = plsc.load_gather(ids_v, [src])
                idx_v[par][pl.ds(li * BB + jg * LANES, LANES)] = val

    def issue_wb(ci, par, bh):
        l0 = ci * LC
        pltpu.async_copy(
            out_v[par],
            out_hbm.at[pl.ds(l0, LC), :, wid, :, pl.ds(bh * BB, BB)],
            sem_w[par],
        )

    def wait_wb(par):
        pltpu.make_async_copy(
            out_v[par],
            out_hbm.at[pl.ds(0, LC), :, wid, :, pl.ds(0, BB)],
            sem_w[par],
        ).wait()

    def compute(ci, par):
        l0 = ci * LC
        rows = rows_v[par]
        outb = out_v[par]

        @pl.loop(0, CHUNK // LANES)
        def _group(g):
            li = g >> 2
            jg = g & 3
            l = l0 + li
            tok = g * LANES + lanes
            tag16 = plsc.load_gather(tags_v, [j200 + (jg * LANES * L + l)])
            cflat = tag16 * (L * D) + l * D

            PRE = 4
            vt = [None] * D
            vc = [None] * D
            rots = [None] * D

            def _issue(d):
                rot = lanes + d
                if d + LANES > D:
                    rot = rot & (D - 1)
                rots[d] = rot
                vt[d] = plsc.load_gather(rows, [tok, rot])
                vc[d] = plsc.load_gather(comb_v, [cflat + rot])

            for d in range(PRE):
                _issue(d)
            acc = jnp.zeros((LANES,), jnp.float32)
            acc2 = jnp.zeros((LANES,), jnp.float32)
            for d in range(D):
                if d + PRE < D:
                    _issue(d + PRE)
                v = vt[d] + vc[d]
                plsc.store_scatter(tbuf_v, [(rots[d] << 4) + lanes], v)
                acc = acc + v
                acc2 = acc2 + v * v

            mean = acc * (1.0 / D)
            var = acc2 * (1.0 / D) - mean * mean
            rstd = _rsqrt_sc(var + EPS)
            mrs = mean * rstd

            for d in range(D):
                k, j = d // LANES, d % LANES
                gd = bcast(gvs[k], j)
                bd = bcast(bvs[k], j)
                y = tbuf_v[pl.ds(d * LANES, LANES)]
                outv = (y * rstd - mrs) * gd + bd
                outb[li, d // 8, d % 8, pl.ds(jg * LANES, LANES)] = outv

    for bh in range(2):
        boff = (wid * 2 + bh) * BB * L
        pltpu.sync_copy(ids_hbm.at[pl.ds(boff, BB * L)], ids_v)
        pltpu.sync_copy(tags_hbm.at[pl.ds(boff, BB * L)], tags_v)

        build_idx(0, 0)
        issue_gather(0)

        @pl.loop(0, NCHUNKS // 2)
        def _super(sc_i):
            for par in range(2):
                ci = sc_i * 2 + par
                wait_gather(par)

                @pl.when(ci < NCHUNKS - 1)
                def _():
                    build_idx(ci + 1, 1 - par)
                    issue_gather(1 - par)

                if bh == 0:
                    @pl.when(ci >= 2)
                    def _():
                        wait_wb(par)
                else:
                    wait_wb(par)

                compute(ci, par)
                issue_wb(ci, par, bh)

    wait_wb(0)
    wait_wb(1)


def kernel(input_ids, is_buyer_tags, token_table, buyer_table, ln_gamma, ln_beta):
    info = plsc.get_sparse_core_info()
    nc = info.num_cores

    table2 = jnp.pad(token_table, ((0, 0), (0, D))).reshape(2 * VOCAB, D)
    ids_flat = input_ids.reshape(N).astype(jnp.int32) * 2
    tags_flat = is_buyer_tags.reshape(N).astype(jnp.int32)
    pe = _positional_encoding(D, L)
    comb = (pe[None, :, :] + buyer_table[:, None, :]).reshape(2 * L * D)

    mesh = plsc.VectorSubcoreMesh(core_axis_name="c", subcore_axis_name="s")
    run = pl.kernel(
        functools.partial(_sc_body, nc),
        out_type=jax.ShapeDtypeStruct((L, D // 8, B // 128, 8, 128), jnp.float32),
        mesh=mesh,
        scratch_types=[
            pltpu.VMEM((BB * L,), jnp.int32),
            pltpu.VMEM((BB * L,), jnp.int32),
            [pltpu.VMEM((CHUNK,), jnp.int32)] * 2,
            [pltpu.VMEM((CHUNK, D), jnp.float32)] * 2,
            pltpu.VMEM((2 * L * D,), jnp.float32),
            pltpu.VMEM((D,), jnp.float32),
            pltpu.VMEM((D,), jnp.float32),
            pltpu.VMEM((D * LANES,), jnp.float32),
            [pltpu.VMEM((LC, D // 8, 8, BB), jnp.float32)] * 2,
            [pltpu.SemaphoreType.DMA] * 2,
            [pltpu.SemaphoreType.DMA] * 2,
        ],
        compiler_params=pltpu.CompilerParams(
            use_tc_tiling_on_sc=False, needs_layout_passes=False
        ),
    )
    out5 = run(ids_flat, tags_flat, table2, comb, ln_gamma, ln_beta)
    return out5.transpose(2, 4, 0, 1, 3).reshape(B, L, D)


# speedup vs baseline: 1.3178x; 1.3178x over previous
import functools

import jax
import jax.numpy as jnp
from jax import lax
from jax.experimental import pallas as pl
from jax.experimental.pallas import tpu as pltpu
from jax.experimental.pallas import tpu_sc as plsc

VOCAB = 1000000
D = 64
B = 4096
L = 200
N = B * L
LANES = 16
BB = 64
LC = 4
CHUNK = LC * BB
NCHUNKS = L // LC
EPS = 1e-5


def _positional_encoding(d_model, max_len):
    pos = jnp.arange(max_len, dtype=jnp.float32)[:, None]
    div = jnp.exp(
        jnp.arange(0, d_model, 2, dtype=jnp.float32) * (-jnp.log(10000.0) / d_model)
    )
    ang = pos * div[None, :]
    pe = jnp.zeros((max_len, d_model), dtype=jnp.float32)
    pe = pe.at[:, 0::2].set(jnp.sin(ang))
    pe = pe.at[:, 1::2].set(jnp.cos(ang))
    return pe


def _rsqrt_sc(x):
    i = lax.bitcast_convert_type(x, jnp.int32)
    i = jnp.int32(0x5F3759DF) - (i >> 1)
    y = lax.bitcast_convert_type(i, jnp.float32)
    for _ in range(2):
        y = y * (1.5 - 0.5 * x * y * y)
    return y


def _sc_body(nc, ids_hbm, tags_hbm, table_hbm, comb_hbm, gam_hbm, bet_hbm,
             out_hbm, ids_v, tags_v, idx_v, rows_v, comb_v, gam_v, bet_v,
             tbuf_v, out_v, sem_g, sem_w):
    wid = lax.axis_index("s") * nc + lax.axis_index("c")

    pltpu.sync_copy(comb_hbm, comb_v)
    pltpu.sync_copy(gam_hbm, gam_v)
    pltpu.sync_copy(bet_hbm, bet_v)

    gvs = [gam_v[pl.ds(k * LANES, LANES)] for k in range(D // LANES)]
    bvs = [bet_v[pl.ds(k * LANES, LANES)] for k in range(D // LANES)]

    lanes = lax.iota(jnp.int32, LANES)
    j200 = lanes * L

    def bcast(v, j):
        return v.at[jnp.full((LANES,), j, jnp.int32)].get(mode="promise_in_bounds")

    def issue_gather(par):
        pltpu.async_copy(table_hbm.at[idx_v[par]], rows_v[par], sem_g[par])

    def wait_gather(par):
        pltpu.make_async_copy(table_hbm.at[idx_v[par]], rows_v[par], sem_g[par]).wait()

    def build_idx(ci, par):
        l0 = ci * LC
        for li in range(LC):
            for jg in range(BB // LANES):
                src = j200 + (jg * LANES * L + l0 + li)
                val = plsc.load_gather(ids_v, [src])
                idx_v[par][pl.ds(li * BB + jg * LANES, LANES)] = val

    def issue_wb(ci, par, bh):
        l0 = ci * LC
        pltpu.async_copy(
            out_v[par],
            out_hbm.at[pl.ds(l0, LC), :, wid, :, pl.ds(bh * BB, BB)],
            sem_w[par],
        )

    def wait_wb(par):
        pltpu.make_async_copy(
            out_v[par],
            out_hbm.at[pl.ds(0, LC), :, wid, :, pl.ds(0, BB)],
            sem_w[par],
        ).wait()

    def compute(ci, par):
        l0 = ci * LC
        rows = rows_v[par]
        outb = out_v[par]

        @pl.loop(0, CHUNK // LANES)
        def _group(g):
            li = g >> 2
            jg = g & 3
            l = l0 + li
            tok = g * LANES + lanes
            tag16 = plsc.load_gather(tags_v, [j200 + (jg * LANES * L + l)])
            cflat = tag16 * (L * D) + l * D

            PRE = 4
            vt = [None] * D
            vc = [None] * D
            rots = [None] * D

            def _issue(d):
                rot = lanes + d
                if d + LANES > D:
                    rot = rot & (D - 1)
                rots[d] = rot
                vt[d] = plsc.load_gather(rows, [tok, rot])
                vc[d] = plsc.load_gather(comb_v, [cflat + rot])

            for d in range(PRE):
                _issue(d)
            acc = jnp.zeros((LANES,), jnp.float32)
            acc2 = jnp.zeros((LANES,), jnp.float32)
            for d in range(D):
                if d + PRE < D:
                    _issue(d + PRE)
                v = vt[d] + vc[d]
                plsc.store_scatter(tbuf_v, [(rots[d] << 4) + lanes], v)
                acc = acc + v
                acc2 = acc2 + v * v

            mean = acc * (1.0 / D)
            var = acc2 * (1.0 / D) - mean * mean
            rstd = _rsqrt_sc(var + EPS)
            mrs = mean * rstd

            BLK = 8
            for d0 in range(0, D, BLK):
                dd = range(d0, d0 + BLK)
                ys = [tbuf_v[pl.ds(d * LANES, LANES)] for d in dd]
                gds = [bcast(gvs[d // LANES], d % LANES) for d in dd]
                bds = [bcast(bvs[d // LANES], d % LANES) for d in dd]
                p = [y * rstd for y in ys]
                q = [x - mrs for x in p]
                r = [x * gd for x, gd in zip(q, gds)]
                s = [x + bd for x, bd in zip(r, bds)]
                for i, d in enumerate(dd):
                    outb[li, d // 8, d % 8, pl.ds(jg * LANES, LANES)] = s[i]

    for bh in range(2):
        boff = (wid * 2 + bh) * BB * L
        pltpu.sync_copy(ids_hbm.at[pl.ds(boff, BB * L)], ids_v)
        pltpu.sync_copy(tags_hbm.at[pl.ds(boff, BB * L)], tags_v)

        build_idx(0, 0)
        issue_gather(0)

        @pl.loop(0, NCHUNKS // 2)
        def _super(sc_i):
            for par in range(2):
                ci = sc_i * 2 + par
                wait_gather(par)

                @pl.when(ci < NCHUNKS - 1)
                def _():
                    build_idx(ci + 1, 1 - par)
                    issue_gather(1 - par)

                if bh == 0:
                    @pl.when(ci >= 2)
                    def _():
                        wait_wb(par)
                else:
                    wait_wb(par)

                compute(ci, par)
                issue_wb(ci, par, bh)

    wait_wb(0)
    wait_wb(1)


def kernel(input_ids, is_buyer_tags, token_table, buyer_table, ln_gamma, ln_beta):
    info = plsc.get_sparse_core_info()
    nc = info.num_cores

    table2 = jnp.pad(token_table, ((0, 0), (0, D))).reshape(2 * VOCAB, D)
    ids_flat = input_ids.reshape(N).astype(jnp.int32) * 2
    tags_flat = is_buyer_tags.reshape(N).astype(jnp.int32)
    pe = _positional_encoding(D, L)
    comb = (pe[None, :, :] + buyer_table[:, None, :]).reshape(2 * L * D)

    mesh = plsc.VectorSubcoreMesh(core_axis_name="c", subcore_axis_name="s")
    run = pl.kernel(
        functools.partial(_sc_body, nc),
        out_type=jax.ShapeDtypeStruct((L, D // 8, B // 128, 8, 128), jnp.float32),
        mesh=mesh,
        scratch_types=[
            pltpu.VMEM((BB * L,), jnp.int32),
            pltpu.VMEM((BB * L,), jnp.int32),
            [pltpu.VMEM((CHUNK,), jnp.int32)] * 2,
            [pltpu.VMEM((CHUNK, D), jnp.float32)] * 2,
            pltpu.VMEM((2 * L * D,), jnp.float32),
            pltpu.VMEM((D,), jnp.float32),
            pltpu.VMEM((D,), jnp.float32),
            pltpu.VMEM((D * LANES,), jnp.float32),
            [pltpu.VMEM((LC, D // 8, 8, BB), jnp.float32)] * 2,
            [pltpu.SemaphoreType.DMA] * 2,
            [pltpu.SemaphoreType.DMA] * 2,
        ],
        compiler_params=pltpu.CompilerParams(
            use_tc_tiling_on_sc=False, needs_layout_passes=False
        ),
    )
    out5 = run(ids_flat, tags_flat, table2, comb, ln_gamma, ln_beta)
    return out5.transpose(2, 4, 0, 1, 3).reshape(B, L, D)
